# baseline (speedup 1.0000x reference)
import functools

import jax
import jax.numpy as jnp
from jax import lax
from jax.experimental import pallas as pl
from jax.experimental.pallas import tpu as pltpu
from jax.experimental.pallas import tpu_sc as plsc

N_ROWS = 16384
NC = 1000
VAL = 1001.0
NUM_WORKERS = 32
ROWS_PER_WORKER = N_ROWS // NUM_WORKERS
CHUNK_ROWS = 32
N_CHUNKS = ROWS_PER_WORKER // CHUNK_ROWS


def _fill_ones(buf):
    ones16 = jnp.full((16,), 1.0, jnp.float32)

    def row_body(r, _):
        def col_body(c, _):
            buf[r, pl.ds(c * 16, 16)] = ones16
            return 0

        lax.fori_loop(0, NC // 16, col_body, 0)
        buf[r, pl.ds(NC - 16, 16)] = ones16
        return 0

    lax.fori_loop(0, CHUNK_ROWS, row_body, 0)


def _poke(buf, lab, chunk, value):
    iota = lax.iota(jnp.int32, 16)
    vals = jnp.full((16,), value, jnp.float32)
    for v in range(CHUNK_ROWS // 16):
        labv = lab[pl.ds(chunk * CHUNK_ROWS + v * 16, 16)]
        plsc.store_scatter(buf, [iota + v * 16, labv], vals)


@functools.partial(
    pl.kernel,
    out_type=jax.ShapeDtypeStruct((N_ROWS, NC), jnp.float32),
    mesh=plsc.VectorSubcoreMesh(core_axis_name="c", subcore_axis_name="s"),
    compiler_params=pltpu.CompilerParams(needs_layout_passes=False),
    scratch_types=[
        pltpu.VMEM((CHUNK_ROWS, NC), jnp.float32),
        pltpu.VMEM((CHUNK_ROWS, NC), jnp.float32),
        pltpu.VMEM((ROWS_PER_WORKER,), jnp.int32),
        pltpu.SemaphoreType.DMA,
        pltpu.SemaphoreType.DMA,
    ],
)
def _sc_smooth_onehot(labels_hbm, out_hbm, buf0, buf1, lab, sem0, sem1):
    wid = lax.axis_index("s") * 2 + lax.axis_index("c")
    row0 = pl.multiple_of(wid * ROWS_PER_WORKER, 8)

    pltpu.sync_copy(labels_hbm.at[pl.ds(row0, ROWS_PER_WORKER)], lab)

    bufs = (buf0, buf1)
    sems = (sem0, sem1)
    copies = [None, None]

    for k in range(N_CHUNKS):
        b = k % 2
        if k < 2:
            _fill_ones(bufs[b])
        else:
            copies[b].wait()
            _poke(bufs[b], lab, k - 2, 1.0)
        _poke(bufs[b], lab, k, VAL)
        dst = out_hbm.at[pl.ds(pl.multiple_of(row0 + k * CHUNK_ROWS, 8),
                               CHUNK_ROWS), :]
        copies[b] = pltpu.async_copy(bufs[b], dst, sems[b])

    copies[0].wait()
    copies[1].wait()


def kernel(labels):
    return _sc_smooth_onehot(labels.astype(jnp.int32))


# speedup vs baseline: 1.0016x; 1.0016x over previous
import functools

import jax
import jax.numpy as jnp
from jax import lax
from jax.experimental import pallas as pl
from jax.experimental.pallas import tpu as pltpu
from jax.experimental.pallas import tpu_sc as plsc

N_ROWS = 16384
NC = 1000
VAL = 1001.0
NUM_WORKERS = 32
ROWS_PER_WORKER = N_ROWS // NUM_WORKERS
CHUNK_ROWS = 32
N_CHUNKS = ROWS_PER_WORKER // CHUNK_ROWS


def _fill_ones(buf):
    ones16 = jnp.full((16,), 1.0, jnp.float32)

    def row_body(r, _):
        def col_body(c, _):
            buf[r, pl.ds(c * 16, 16)] = ones16
            return 0

        lax.fori_loop(0, NC // 16, col_body, 0)
        buf[r, pl.ds(NC - 16, 16)] = ones16
        return 0

    lax.fori_loop(0, CHUNK_ROWS, row_body, 0)


def _poke(buf, lab, chunk, value):
    iota = lax.iota(jnp.int32, 16)
    vals = jnp.full((16,), value, jnp.float32)
    for v in range(CHUNK_ROWS // 16):
        labv = lab[pl.ds(chunk * CHUNK_ROWS + v * 16, 16)]
        plsc.store_scatter(buf, [iota + v * 16, labv], vals)


@functools.partial(
    pl.kernel,
    out_type=jax.ShapeDtypeStruct((N_ROWS, NC), jnp.float32),
    mesh=plsc.VectorSubcoreMesh(core_axis_name="c", subcore_axis_name="s"),
    compiler_params=pltpu.CompilerParams(
        needs_layout_passes=False, use_tc_tiling_on_sc=True),
    scratch_types=[
        pltpu.VMEM((CHUNK_ROWS, NC), jnp.float32),
        pltpu.VMEM((CHUNK_ROWS, NC), jnp.float32),
        pltpu.VMEM((ROWS_PER_WORKER,), jnp.int32),
        pltpu.SemaphoreType.DMA,
        pltpu.SemaphoreType.DMA,
    ],
)
def _sc_smooth_onehot(labels_hbm, out_hbm, buf0, buf1, lab, sem0, sem1):
    wid = lax.axis_index("s") * 2 + lax.axis_index("c")
    row0 = pl.multiple_of(wid * ROWS_PER_WORKER, 8)

    pltpu.sync_copy(labels_hbm.at[pl.ds(row0, ROWS_PER_WORKER)], lab)

    bufs = (buf0, buf1)
    sems = (sem0, sem1)
    copies = [None, None]

    for k in range(N_CHUNKS):
        b = k % 2
        if k < 2:
            _fill_ones(bufs[b])
        else:
            copies[b].wait()
            _poke(bufs[b], lab, k - 2, 1.0)
        _poke(bufs[b], lab, k, VAL)
        dst = out_hbm.at[pl.ds(pl.multiple_of(row0 + k * CHUNK_ROWS, 8),
                               CHUNK_ROWS), :]
        copies[b] = pltpu.async_copy(bufs[b], dst, sems[b])

    copies[0].wait()
    copies[1].wait()


def kernel(labels):
    return _sc_smooth_onehot(labels.astype(jnp.int32))


# speedup vs baseline: 5.0536x; 5.0454x over previous
import jax
import jax.numpy as jnp
from jax.experimental import pallas as pl

N_ROWS = 16384
NC = 1000
VAL = 1001.0
COLS_PER_BLOCK = 2048


def _smooth_onehot_t_block(lab_ref, out_ref):
    lab = lab_ref[...]
    jrow = jax.lax.broadcasted_iota(jnp.int32, (NC, lab.shape[1]), 0)
    out_ref[...] = jnp.where(lab == jrow, VAL, 1.0)


def kernel(labels):
    c = COLS_PER_BLOCK
    lab2d = labels.astype(jnp.int32).reshape(1, N_ROWS)
    out_t = pl.pallas_call(
        _smooth_onehot_t_block,
        grid=(N_ROWS // c,),
        in_specs=[pl.BlockSpec((1, c), lambda i: (0, i))],
        out_specs=pl.BlockSpec((NC, c), lambda i: (0, i)),
        out_shape=jax.ShapeDtypeStruct((NC, N_ROWS), jnp.float32),
    )(lab2d)
    return out_t.T
